# baseline (speedup 1.0000x reference)
import functools

import jax
import jax.numpy as jnp
from jax import lax
from jax.experimental import pallas as pl
from jax.experimental.pallas import tpu as pltpu
from jax.experimental.pallas import tpu_sc as plsc

L = 16
ROWS = 32
T = 8
V = 100000
NBLK = V // L


def _argmax_kernel(x_hbm, out_hbm, row_v, res_v, sem):
    nc = 2
    wid = lax.axis_index("s") * nc + lax.axis_index("c")
    row = wid * T + (T - 1)
    pltpu.async_copy(x_hbm.at[row], row_v, sem).wait()

    def body(i, carry):
        mx, ix = carry
        v = row_v[pl.ds(i * L, L)]
        pred = v > mx
        return jnp.where(pred, v, mx), jnp.where(pred, i, ix)

    mx0 = row_v[pl.ds(0, L)]
    ix0 = jnp.zeros((L,), jnp.int32)
    mx, ix = lax.fori_loop(1, NBLK, body, (mx0, ix0))

    iota = lax.iota(jnp.int32, L)

    def shuffle(v, s):
        return v.at[iota ^ s].get(mode="promise_in_bounds")

    gi = ix * L + iota
    m = mx
    for s in (8, 4, 2, 1):
        m = jnp.maximum(m, shuffle(m, s))
    cand = jnp.where(mx == m, gi, jnp.int32(2**31 - 1))
    for s in (8, 4, 2, 1):
        cand = jnp.minimum(cand, shuffle(cand, s))
    res_v[...] = cand
    pltpu.sync_copy(res_v, out_hbm.at[wid])


def kernel(measure):
    x = measure.reshape(ROWS * T, V)
    mesh = plsc.VectorSubcoreMesh(core_axis_name="c", subcore_axis_name="s")
    run = functools.partial(
        pl.kernel,
        mesh=mesh,
        out_type=jax.ShapeDtypeStruct((ROWS, L), jnp.int32),
        scratch_types=[
            pltpu.VMEM((V,), jnp.float32),
            pltpu.VMEM((L,), jnp.int32),
            pltpu.SemaphoreType.DMA,
        ],
    )(_argmax_kernel)
    out = run(x)
    return out[:, 0]
